# baseline (speedup 1.0000x reference)
import functools

import jax
import jax.numpy as jnp
from jax import lax
from jax.experimental import pallas as pl
from jax.experimental.pallas import tpu as pltpu
from jax.experimental.pallas import tpu_sc as plsc

_N = 16384
_D = 2048
_E = 64
_R = 2048

_NW = 32
_C = _N // _NW
_L = 16
_G = _C // _L

def _keys_block(x_ref, w_ref, b_ref, out_ref):
    logits = lax.dot_general(
        w_ref[...], x_ref[...],
        dimension_numbers=(((0,), (1,)), ((), ())),
        preferred_element_type=jnp.float32,
    ) + b_ref[...]
    bits = lax.bitcast_convert_type(logits, jnp.int32)
    s = bits ^ (lax.shift_right_arithmetic(bits, 31) & 0x7FFFFFFF)
    erow = lax.broadcasted_iota(jnp.int32, logits.shape, 0)
    out_ref[...] = (s & ~63) | (63 - erow)


def _keys_T(x, w, b):
    return pl.pallas_call(
        _keys_block,
        grid=(_N // _R,),
        in_specs=[
            pl.BlockSpec((_R, _D), lambda i: (i, 0)),
            pl.BlockSpec((_D, _E), lambda i: (0, 0)),
            pl.BlockSpec((_E, 1), lambda i: (0, 0)),
        ],
        out_specs=pl.BlockSpec((_E, _R), lambda i: (0, i)),
        out_shape=jax.ShapeDtypeStruct((_E, _N), jnp.int32),
        compiler_params=pltpu.CompilerParams(
            dimension_semantics=("arbitrary",),
        ),
    )(x, w, b.reshape(_E, 1))


def _key_to_logit(key):
    s = key & ~63
    bits = s ^ (lax.shift_right_arithmetic(s, 31) & 0x7FFFFFFF)
    return lax.bitcast_convert_type(bits, jnp.float32)


@functools.partial(
    pl.kernel,
    out_type=[
        jax.ShapeDtypeStruct((2, _N), jnp.int32),
        jax.ShapeDtypeStruct((2, _N), jnp.float32),
    ],
    mesh=plsc.VectorSubcoreMesh(core_axis_name="c", subcore_axis_name="s"),
    scratch_types=[
        pltpu.VMEM((_E, _C), jnp.int32),
        pltpu.VMEM((_C,), jnp.int32),
        pltpu.VMEM((_C,), jnp.int32),
        pltpu.VMEM((_C,), jnp.float32),
        pltpu.VMEM((_C,), jnp.float32),
        pltpu.SemaphoreType.DMA,
        pltpu.SemaphoreType.DMA,
        pltpu.SemaphoreType.DMA,
        pltpu.SemaphoreType.DMA,
    ],
)
def _sc_top2(keys_hbm, idx_hbm, probs_hbm, kbuf, i1b, i2b, p1b, p2b,
             sem0, sem1, sem2, sem3):
    wid = lax.axis_index("s") * 2 + lax.axis_index("c")
    base = wid * _C
    nslc = 4
    cslc = _C // nslc
    sems = (sem0, sem1, sem2, sem3)
    copies = [
        pltpu.async_copy(
            keys_hbm.at[:, pl.ds(base + s * cslc, cslc)],
            kbuf.at[:, pl.ds(s * cslc, cslc)],
            sems[s],
        )
        for s in range(nslc)
    ]

    def group(g, carry):
        sl = pl.ds(g * _L, _L)
        m1 = kbuf[0, sl]
        m2 = jnp.full((_L,), jnp.iinfo(jnp.int32).min, jnp.int32)
        for e in range(1, _E):
            k = kbuf[e, sl]
            m2 = jnp.maximum(m2, jnp.minimum(m1, k))
            m1 = jnp.maximum(m1, k)
        i1b[sl] = 63 - (m1 & 63)
        i2b[sl] = 63 - (m2 & 63)
        ex = jnp.exp(_key_to_logit(m2) - _key_to_logit(m1))
        den = 1.0 + ex
        p1b[sl] = 1.0 / den
        p2b[sl] = ex / den
        return carry

    gslc = _G // nslc
    for s in range(nslc):
        copies[s].wait()
        lax.fori_loop(s * gslc, (s + 1) * gslc, group, 0)
    pltpu.sync_copy(i1b, idx_hbm.at[0, pl.ds(base, _C)])
    pltpu.sync_copy(i2b, idx_hbm.at[1, pl.ds(base, _C)])
    pltpu.sync_copy(p1b, probs_hbm.at[0, pl.ds(base, _C)])
    pltpu.sync_copy(p2b, probs_hbm.at[1, pl.ds(base, _C)])


def kernel(x, W_gate, b_gate):
    keys_t = _keys_T(x, W_gate, b_gate)
    idx_t, probs_t = _sc_top2(keys_t)
    return (idx_t.T, probs_t.T)


# speedup vs baseline: 1.0103x; 1.0103x over previous
import functools

import jax
import jax.numpy as jnp
from jax import lax
from jax.experimental import pallas as pl
from jax.experimental.pallas import tpu as pltpu
from jax.experimental.pallas import tpu_sc as plsc

_N = 16384
_D = 2048
_E = 64
_R = 2048

_NW = 32
_C = _N // _NW
_L = 16
_G = _C // _L

def _keys_block(x_ref, w_ref, b_ref, out_ref):
    logits = lax.dot_general(
        w_ref[...], x_ref[...],
        dimension_numbers=(((0,), (1,)), ((), ())),
        preferred_element_type=jnp.float32,
    ) + b_ref[...]
    bits = lax.bitcast_convert_type(logits, jnp.int32)
    s = bits ^ (lax.shift_right_arithmetic(bits, 31) & 0x7FFFFFFF)
    erow = lax.broadcasted_iota(jnp.int32, logits.shape, 0)
    out_ref[...] = (s & ~63) | (63 - erow)


def _keys_T(x, w, b):
    return pl.pallas_call(
        _keys_block,
        grid=(_N // _R,),
        in_specs=[
            pl.BlockSpec((_R, _D), lambda i: (i, 0)),
            pl.BlockSpec((_D, _E), lambda i: (0, 0)),
            pl.BlockSpec((_E, 1), lambda i: (0, 0)),
        ],
        out_specs=pl.BlockSpec((_E, _R), lambda i: (0, i)),
        out_shape=jax.ShapeDtypeStruct((_E, _N), jnp.int32),
        compiler_params=pltpu.CompilerParams(
            dimension_semantics=("arbitrary",),
        ),
    )(x, w, b.reshape(_E, 1))


def _key_to_logit(key):
    s = key & ~63
    bits = s ^ (lax.shift_right_arithmetic(s, 31) & 0x7FFFFFFF)
    return lax.bitcast_convert_type(bits, jnp.float32)


@functools.partial(
    pl.kernel,
    out_type=[
        jax.ShapeDtypeStruct((2, _N), jnp.int32),
        jax.ShapeDtypeStruct((2, _N), jnp.float32),
    ],
    mesh=plsc.VectorSubcoreMesh(core_axis_name="c", subcore_axis_name="s"),
    scratch_types=[
        pltpu.VMEM((_E, _C), jnp.int32),
        pltpu.VMEM((_C,), jnp.int32),
        pltpu.VMEM((_C,), jnp.int32),
        pltpu.VMEM((_C,), jnp.float32),
        pltpu.VMEM((_C,), jnp.float32),
    ],
)
def _sc_top2(keys_hbm, idx_hbm, probs_hbm, kbuf, i1b, i2b, p1b, p2b):
    wid = lax.axis_index("s") * 2 + lax.axis_index("c")
    base = wid * _C
    pltpu.sync_copy(keys_hbm.at[:, pl.ds(base, _C)], kbuf)

    def group(g, carry):
        sl = pl.ds(g * _L, _L)
        m1 = kbuf[0, sl]
        m2 = jnp.full((_L,), jnp.iinfo(jnp.int32).min, jnp.int32)
        for e in range(1, _E):
            k = kbuf[e, sl]
            m2 = jnp.maximum(m2, jnp.minimum(m1, k))
            m1 = jnp.maximum(m1, k)
        i1b[sl] = 63 - (m1 & 63)
        i2b[sl] = 63 - (m2 & 63)
        ex = jnp.exp(_key_to_logit(m2) - _key_to_logit(m1))
        den = 1.0 + ex
        p1b[sl] = 1.0 / den
        p2b[sl] = ex / den
        return carry

    lax.fori_loop(0, _G, group, 0)
    pltpu.sync_copy(i1b, idx_hbm.at[0, pl.ds(base, _C)])
    pltpu.sync_copy(i2b, idx_hbm.at[1, pl.ds(base, _C)])
    pltpu.sync_copy(p1b, probs_hbm.at[0, pl.ds(base, _C)])
    pltpu.sync_copy(p2b, probs_hbm.at[1, pl.ds(base, _C)])


def kernel(x, W_gate, b_gate):
    keys_t = _keys_T(x, W_gate, b_gate)
    idx_t, probs_t = _sc_top2(keys_t)
    return (idx_t.T, probs_t.T)
